# baseline (speedup 1.0000x reference)
import functools

import jax
import jax.numpy as jnp
from jax import lax
from jax.experimental import pallas as pl
from jax.experimental.pallas import tpu as pltpu


def _build_xcat(x_ref, dst_ref, masks, *, C, W, HW):
    bf16 = jnp.bfloat16
    xb = x_ref[0].astype(bf16).reshape(C, HW)

    for t in range(9):
        dy, dx = divmod(t, 3)
        off = ((1 - dy) * W + (1 - dx)) % HW
        blk = xb if off == 0 else pltpu.roll(xb, off, axis=1)
        if t != 4:
            blk = blk * masks[t]
        dst_ref[t * C:(t + 1) * C, :] = blk


def _attend(src_ref, w3_ref, wproj_ref, bproj_ref, temp_ref, o_ref,
            *, C, H, W, HW, num_heads):
    f32 = jnp.float32
    bf16 = jnp.bfloat16
    g = jnp.dot(w3_ref[...], src_ref[...],
                preferred_element_type=f32)
    gb = g.astype(bf16)
    q = gb[0:C]
    k = gb[C:2 * C]
    v = gb[2 * C:3 * C]
    qf = g[0:C]
    kf = g[C:2 * C]

    ones_row = jnp.ones((1, HW), f32)
    sq = lax.dot_general(qf * qf, ones_row, (((1,), (1,)), ((), ())),
                         preferred_element_type=f32)
    sk = lax.dot_general(ones_row, kf * kf, (((1,), (1,)), ((), ())),
                         preferred_element_type=f32)
    gram = lax.dot_general(q, k, (((1,), (1,)), ((), ())),
                           preferred_element_type=f32)

    eps2 = 1e-24
    qn = lax.rsqrt(jnp.maximum(sq, eps2))
    kn = lax.rsqrt(jnp.maximum(sk, eps2))
    sc = gram * qn * kn * temp_ref[...]

    hc = C // num_heads
    ri = lax.broadcasted_iota(jnp.int32, (C, C), 0)
    ci = lax.broadcasted_iota(jnp.int32, (C, C), 1)
    sc = jnp.where((ri // hc) == (ci // hc), sc, -1e30)
    sc = sc - jnp.max(sc, axis=-1, keepdims=True)
    e = jnp.exp(sc)
    p = e / jnp.sum(e, axis=-1, keepdims=True)

    m = lax.dot_general(p, wproj_ref[...], (((0,), (0,)), ((), ())),
                        preferred_element_type=f32)
    out = lax.dot_general(m.astype(bf16), v, (((0,), (0,)), ((), ())),
                          preferred_element_type=f32)
    o_ref[0] = (out + bproj_ref[...]).reshape(C, H, W)


def _fused_attention_kernel(x_ref, w3_ref, wproj_ref, bproj_ref, temp_ref,
                            o_ref, xcat0_ref, xcat1_ref,
                            *, H, W, num_heads):
    bf16 = jnp.bfloat16
    C = x_ref.shape[1]
    HW = H * W
    s = pl.program_id(0)

    pos = lax.broadcasted_iota(jnp.int32, (1, HW), 1)
    jcol = pos % W
    yrow = pos // W

    cl = jnp.where(jcol == 0, 0.0, 1.0).astype(bf16)
    cr = jnp.where(jcol == W - 1, 0.0, 1.0).astype(bf16)
    rt = jnp.where(yrow == 0, 0.0, 1.0).astype(bf16)
    rb = jnp.where(yrow == H - 1, 0.0, 1.0).astype(bf16)

    one = jnp.ones((1, HW), bf16)
    masks = [cl * rt, rt, cr * rt,
             cl, one, cr,
             cl * rb, rb, cr * rb]

    @pl.when(s == 0)
    def _():
        zrow = jnp.zeros((6, HW), bf16)
        indcat = jnp.concatenate(masks + [one, zrow], axis=0)
        xcat0_ref[9 * C:, :] = indcat
        xcat1_ref[9 * C:, :] = indcat
        xcat1_ref[0:9 * C, :] = jnp.zeros((9 * C, HW), bf16)

    bk = dict(C=C, W=W, HW=HW)
    ak = dict(C=C, H=H, W=W, HW=HW, num_heads=num_heads)

    @pl.when(s % 2 == 0)
    def _():
        _attend(xcat1_ref, w3_ref, wproj_ref, bproj_ref, temp_ref,
                o_ref, **ak)
        _build_xcat(x_ref, xcat0_ref, masks, **bk)

    @pl.when(s % 2 == 1)
    def _():
        _attend(xcat0_ref, w3_ref, wproj_ref, bproj_ref, temp_ref,
                o_ref, **ak)
        _build_xcat(x_ref, xcat1_ref, masks, **bk)


def kernel(x, temperature, w_qkv, b_qkv, w_dw, b_dw, w_proj, b_proj):
    B, C, H, W = x.shape
    HW = H * W
    num_heads = temperature.shape[0]
    C3 = 3 * C

    wdw9 = w_dw.reshape(9, C3)
    w3 = jnp.einsum('ec,tc->cte', w_qkv, wdw9)
    w3 = w3.reshape(C3, 9 * C)
    wb9 = (b_qkv[None, :] * wdw9).T
    w3_full = jnp.concatenate(
        [w3, wb9, b_dw.reshape(C3, 1), jnp.zeros((C3, 6), w3.dtype)],
        axis=1).astype(jnp.bfloat16)

    bproj_c = b_proj.reshape(C, 1)
    temp_col = jnp.repeat(temperature.astype(jnp.float32),
                          C // num_heads).reshape(C, 1)

    body = functools.partial(_fused_attention_kernel,
                             H=H, W=W, num_heads=num_heads)
    K = 9 * C + 16
    out = pl.pallas_call(
        body,
        out_shape=jax.ShapeDtypeStruct((B, C, H, W), jnp.float32),
        grid=(B + 1,),
        in_specs=[
            pl.BlockSpec((1, C, H, W),
                         lambda b: (jnp.minimum(b, B - 1), 0, 0, 0)),
            pl.BlockSpec((C3, K), lambda b: (0, 0)),
            pl.BlockSpec((C, C), lambda b: (0, 0)),
            pl.BlockSpec((C, 1), lambda b: (0, 0)),
            pl.BlockSpec((C, 1), lambda b: (0, 0)),
        ],
        out_specs=pl.BlockSpec((1, C, H, W),
                               lambda b: (jnp.maximum(b - 1, 0), 0, 0, 0)),
        scratch_shapes=[pltpu.VMEM((K, HW), jnp.bfloat16),
                        pltpu.VMEM((K, HW), jnp.bfloat16)],
        compiler_params=pltpu.CompilerParams(
            dimension_semantics=("arbitrary",),
            vmem_limit_bytes=64 * 1024 * 1024,
        ),
    )(x, w3_full, w_proj, bproj_c, temp_col)
    return out


# speedup vs baseline: 1.0239x; 1.0239x over previous
import functools

import jax
import jax.numpy as jnp
from jax import lax
from jax.experimental import pallas as pl
from jax.experimental.pallas import tpu as pltpu


def _build_xcat(x_ref, dst_ref, masks, *, C, W, HW):
    bf16 = jnp.bfloat16
    xb = x_ref[0].astype(bf16).reshape(C, HW)

    for t in range(9):
        dy, dx = divmod(t, 3)
        off = ((1 - dy) * W + (1 - dx)) % HW
        blk = xb if off == 0 else pltpu.roll(xb, off, axis=1)
        if t != 4:
            blk = blk * masks[t]
        dst_ref[t * C:(t + 1) * C, :] = blk


def _attend(src_ref, w3_ref, wproj_ref, bproj_ref, temp_ref, o_ref,
            *, C, H, W, HW, num_heads):
    f32 = jnp.float32
    bf16 = jnp.bfloat16
    g = jnp.dot(w3_ref[...], src_ref[...],
                preferred_element_type=f32)
    q = g[0:C]
    k = g[C:2 * C]
    v = g[2 * C:3 * C]

    ones_row = jnp.ones((1, HW), f32)
    sq = lax.dot_general(q * q, ones_row, (((1,), (1,)), ((), ())),
                         preferred_element_type=f32)
    sk = lax.dot_general(ones_row, k * k, (((1,), (1,)), ((), ())),
                         preferred_element_type=f32)
    gram = lax.dot_general(q, k, (((1,), (1,)), ((), ())),
                           preferred_element_type=f32)

    eps2 = 1e-24
    qn = lax.rsqrt(jnp.maximum(sq, eps2))
    kn = lax.rsqrt(jnp.maximum(sk, eps2))
    sc = gram * qn * kn * temp_ref[...]

    hc = C // num_heads
    ri = lax.broadcasted_iota(jnp.int32, (C, C), 0)
    ci = lax.broadcasted_iota(jnp.int32, (C, C), 1)
    sc = jnp.where((ri // hc) == (ci // hc), sc, -1e30)
    sc = sc - jnp.max(sc, axis=-1, keepdims=True)
    e = jnp.exp(sc)
    p = e / jnp.sum(e, axis=-1, keepdims=True)

    m = lax.dot_general(p, wproj_ref[...], (((0,), (0,)), ((), ())),
                        preferred_element_type=f32)
    out = lax.dot_general(m, v, (((0,), (0,)), ((), ())),
                          preferred_element_type=f32)
    o_ref[0] = (out + bproj_ref[...]).reshape(C, H, W)


def _fused_attention_kernel(x_ref, w3_ref, wproj_ref, bproj_ref, temp_ref,
                            o_ref, xcat0_ref, xcat1_ref,
                            *, H, W, num_heads):
    bf16 = jnp.bfloat16
    C = x_ref.shape[1]
    HW = H * W
    s = pl.program_id(0)

    pos = lax.broadcasted_iota(jnp.int32, (1, HW), 1)
    jcol = pos % W
    yrow = pos // W

    cl = jnp.where(jcol == 0, 0.0, 1.0).astype(bf16)
    cr = jnp.where(jcol == W - 1, 0.0, 1.0).astype(bf16)
    rt = jnp.where(yrow == 0, 0.0, 1.0).astype(bf16)
    rb = jnp.where(yrow == H - 1, 0.0, 1.0).astype(bf16)

    one = jnp.ones((1, HW), bf16)
    masks = [cl * rt, rt, cr * rt,
             cl, one, cr,
             cl * rb, rb, cr * rb]

    @pl.when(s == 0)
    def _():
        zrow = jnp.zeros((6, HW), bf16)
        indcat = jnp.concatenate(masks + [one, zrow], axis=0)
        xcat0_ref[9 * C:, :] = indcat
        xcat1_ref[9 * C:, :] = indcat
        xcat1_ref[0:9 * C, :] = jnp.zeros((9 * C, HW), bf16)

    bk = dict(C=C, W=W, HW=HW)
    ak = dict(C=C, H=H, W=W, HW=HW, num_heads=num_heads)

    @pl.when(s % 2 == 0)
    def _():
        _attend(xcat1_ref, w3_ref, wproj_ref, bproj_ref, temp_ref,
                o_ref, **ak)
        _build_xcat(x_ref, xcat0_ref, masks, **bk)

    @pl.when(s % 2 == 1)
    def _():
        _attend(xcat0_ref, w3_ref, wproj_ref, bproj_ref, temp_ref,
                o_ref, **ak)
        _build_xcat(x_ref, xcat1_ref, masks, **bk)


def kernel(x, temperature, w_qkv, b_qkv, w_dw, b_dw, w_proj, b_proj):
    B, C, H, W = x.shape
    HW = H * W
    num_heads = temperature.shape[0]
    C3 = 3 * C

    wdw9 = w_dw.reshape(9, C3)
    w3 = jnp.einsum('ec,tc->cte', w_qkv, wdw9)
    w3 = w3.reshape(C3, 9 * C)
    wb9 = (b_qkv[None, :] * wdw9).T
    w3_full = jnp.concatenate(
        [w3, wb9, b_dw.reshape(C3, 1), jnp.zeros((C3, 6), w3.dtype)],
        axis=1).astype(jnp.bfloat16)

    bproj_c = b_proj.reshape(C, 1)
    temp_col = jnp.repeat(temperature.astype(jnp.float32),
                          C // num_heads).reshape(C, 1)

    body = functools.partial(_fused_attention_kernel,
                             H=H, W=W, num_heads=num_heads)
    K = 9 * C + 16
    out = pl.pallas_call(
        body,
        out_shape=jax.ShapeDtypeStruct((B, C, H, W), jnp.float32),
        grid=(B + 1,),
        in_specs=[
            pl.BlockSpec((1, C, H, W),
                         lambda b: (jnp.minimum(b, B - 1), 0, 0, 0)),
            pl.BlockSpec((C3, K), lambda b: (0, 0)),
            pl.BlockSpec((C, C), lambda b: (0, 0)),
            pl.BlockSpec((C, 1), lambda b: (0, 0)),
            pl.BlockSpec((C, 1), lambda b: (0, 0)),
        ],
        out_specs=pl.BlockSpec((1, C, H, W),
                               lambda b: (jnp.maximum(b - 1, 0), 0, 0, 0)),
        scratch_shapes=[pltpu.VMEM((K, HW), jnp.bfloat16),
                        pltpu.VMEM((K, HW), jnp.bfloat16)],
        compiler_params=pltpu.CompilerParams(
            dimension_semantics=("arbitrary",),
            vmem_limit_bytes=64 * 1024 * 1024,
        ),
    )(x, w3_full, w_proj, bproj_c, temp_col)
    return out


# speedup vs baseline: 1.0680x; 1.0431x over previous
import functools

import jax
import jax.numpy as jnp
from jax import lax
from jax.experimental import pallas as pl
from jax.experimental.pallas import tpu as pltpu


def _build_xcat(x_ref, i, dst_ref, cl, cr, rt, rb, *, C, W, HW):
    bf16 = jnp.bfloat16
    xb = x_ref[i].astype(bf16).reshape(C, HW)
    xl = pltpu.roll(xb, 1, axis=1) * cl
    xr = pltpu.roll(xb, HW - 1, axis=1) * cr

    def down(a):
        return pltpu.roll(a, W, axis=1) * rt

    def up(a):
        return pltpu.roll(a, HW - W, axis=1) * rb

    for t, blk in enumerate([
            down(xl), down(xb), down(xr),
            xl, xb, xr,
            up(xl), up(xb), up(xr)]):
        dst_ref[t * C:(t + 1) * C, :] = blk


def _attend(src_ref, w3_ref, wproj_ref, bproj_ref, temp_ref, o_ref, i,
            *, C, H, W, HW, num_heads):
    f32 = jnp.float32
    bf16 = jnp.bfloat16
    g = jnp.dot(w3_ref[...], src_ref[...],
                preferred_element_type=f32)
    q = g[0:C]
    k = g[C:2 * C]
    v = g[2 * C:3 * C]

    ones_row = jnp.ones((1, HW), f32)
    sq = lax.dot_general(q * q, ones_row, (((1,), (1,)), ((), ())),
                         preferred_element_type=f32)
    sk = lax.dot_general(ones_row, k * k, (((1,), (1,)), ((), ())),
                         preferred_element_type=f32)
    gram = lax.dot_general(q, k, (((1,), (1,)), ((), ())),
                           preferred_element_type=f32)

    eps2 = 1e-24
    qn = lax.rsqrt(jnp.maximum(sq, eps2))
    kn = lax.rsqrt(jnp.maximum(sk, eps2))
    sc = gram * qn * kn * temp_ref[...]

    hc = C // num_heads
    ri = lax.broadcasted_iota(jnp.int32, (C, C), 0)
    ci = lax.broadcasted_iota(jnp.int32, (C, C), 1)
    sc = jnp.where((ri // hc) == (ci // hc), sc, -1e30)
    sc = sc - jnp.max(sc, axis=-1, keepdims=True)
    e = jnp.exp(sc)
    p = e / jnp.sum(e, axis=-1, keepdims=True)

    m = lax.dot_general(p, wproj_ref[...], (((0,), (0,)), ((), ())),
                        preferred_element_type=f32)
    out = lax.dot_general(m, v, (((0,), (0,)), ((), ())),
                          preferred_element_type=f32)
    o_ref[i] = (out + bproj_ref[...]).reshape(C, H, W)


def _fused_attention_kernel(x_ref, w3_ref, wproj_ref, bproj_ref, temp_ref,
                            o_ref, e0_ref, e1_ref, o0_ref, o1_ref,
                            *, H, W, num_heads):
    bf16 = jnp.bfloat16
    C = x_ref.shape[1]
    HW = H * W
    s = pl.program_id(0)

    pos = lax.broadcasted_iota(jnp.int32, (1, HW), 1)
    jcol = pos % W
    yrow = pos // W

    cl = jnp.where(jcol == 0, 0.0, 1.0).astype(bf16)
    cr = jnp.where(jcol == W - 1, 0.0, 1.0).astype(bf16)
    rt = jnp.where(yrow == 0, 0.0, 1.0).astype(bf16)
    rb = jnp.where(yrow == H - 1, 0.0, 1.0).astype(bf16)

    one = jnp.ones((1, HW), bf16)
    masks = [cl * rt, rt, cr * rt,
             cl, one, cr,
             cl * rb, rb, cr * rb]

    @pl.when(s == 0)
    def _():
        zrow = jnp.zeros((6, HW), bf16)
        indcat = jnp.concatenate(masks + [one, zrow], axis=0)
        for r in (e0_ref, e1_ref, o0_ref, o1_ref):
            r[9 * C:, :] = indcat
        z = jnp.zeros((9 * C, HW), bf16)
        o0_ref[0:9 * C, :] = z
        o1_ref[0:9 * C, :] = z

    bk = dict(C=C, W=W, HW=HW)
    ak = dict(C=C, H=H, W=W, HW=HW, num_heads=num_heads)

    @pl.when(s % 2 == 0)
    def _():
        _attend(o0_ref, w3_ref, wproj_ref, bproj_ref, temp_ref,
                o_ref, 0, **ak)
        _attend(o1_ref, w3_ref, wproj_ref, bproj_ref, temp_ref,
                o_ref, 1, **ak)
        _build_xcat(x_ref, 0, e0_ref, cl, cr, rt, rb, **bk)
        _build_xcat(x_ref, 1, e1_ref, cl, cr, rt, rb, **bk)

    @pl.when(s % 2 == 1)
    def _():
        _attend(e0_ref, w3_ref, wproj_ref, bproj_ref, temp_ref,
                o_ref, 0, **ak)
        _attend(e1_ref, w3_ref, wproj_ref, bproj_ref, temp_ref,
                o_ref, 1, **ak)
        _build_xcat(x_ref, 0, o0_ref, cl, cr, rt, rb, **bk)
        _build_xcat(x_ref, 1, o1_ref, cl, cr, rt, rb, **bk)


def kernel(x, temperature, w_qkv, b_qkv, w_dw, b_dw, w_proj, b_proj):
    B, C, H, W = x.shape
    HW = H * W
    num_heads = temperature.shape[0]
    C3 = 3 * C

    wdw9 = w_dw.reshape(9, C3)
    w3 = jnp.einsum('ec,tc->cte', w_qkv, wdw9)
    w3 = w3.reshape(C3, 9 * C)
    wb9 = (b_qkv[None, :] * wdw9).T
    w3_full = jnp.concatenate(
        [w3, wb9, b_dw.reshape(C3, 1), jnp.zeros((C3, 6), w3.dtype)],
        axis=1).astype(jnp.bfloat16)

    bproj_c = b_proj.reshape(C, 1)
    temp_col = jnp.repeat(temperature.astype(jnp.float32),
                          C // num_heads).reshape(C, 1)

    body = functools.partial(_fused_attention_kernel,
                             H=H, W=W, num_heads=num_heads)
    K = 9 * C + 16
    NP = B // 2
    out = pl.pallas_call(
        body,
        out_shape=jax.ShapeDtypeStruct((B, C, H, W), jnp.float32),
        grid=(NP + 1,),
        in_specs=[
            pl.BlockSpec((2, C, H, W),
                         lambda b: (jnp.minimum(b, NP - 1), 0, 0, 0)),
            pl.BlockSpec((C3, K), lambda b: (0, 0)),
            pl.BlockSpec((C, C), lambda b: (0, 0)),
            pl.BlockSpec((C, 1), lambda b: (0, 0)),
            pl.BlockSpec((C, 1), lambda b: (0, 0)),
        ],
        out_specs=pl.BlockSpec((2, C, H, W),
                               lambda b: (jnp.maximum(b - 1, 0), 0, 0, 0)),
        scratch_shapes=[pltpu.VMEM((K, HW), jnp.bfloat16),
                        pltpu.VMEM((K, HW), jnp.bfloat16),
                        pltpu.VMEM((K, HW), jnp.bfloat16),
                        pltpu.VMEM((K, HW), jnp.bfloat16)],
        compiler_params=pltpu.CompilerParams(
            dimension_semantics=("arbitrary",),
            vmem_limit_bytes=64 * 1024 * 1024,
        ),
    )(x, w3_full, w_proj, bproj_c, temp_col)
    return out


# speedup vs baseline: 1.0951x; 1.0253x over previous
import functools

import jax
import jax.numpy as jnp
from jax import lax
from jax.experimental import pallas as pl
from jax.experimental.pallas import tpu as pltpu


def _build_xcat(x_ref, i, dst_ref, cl, cr, rt, rb, *, C, W, HW):
    bf16 = jnp.bfloat16
    xb = x_ref[i].astype(bf16).reshape(C, HW)
    xl = pltpu.roll(xb, 1, axis=1) * cl
    xr = pltpu.roll(xb, HW - 1, axis=1) * cr

    def down(a):
        return pltpu.roll(a, W, axis=1) * rt

    def up(a):
        return pltpu.roll(a, HW - W, axis=1) * rb

    for t, blk in enumerate([
            down(xl), down(xb), down(xr),
            xl, xb, xr,
            up(xl), up(xb), up(xr)]):
        dst_ref[t * C:(t + 1) * C, :] = blk


def _attend2(srcA_ref, srcB_ref, w3_ref, wproj_ref, bproj_ref, temp_ref,
             o_ref, *, C, H, W, HW, num_heads):
    f32 = jnp.float32
    gs = [jnp.dot(w3_ref[...], src[...], preferred_element_type=f32)
          for src in (srcA_ref, srcB_ref)]
    qs = [g[0:C] for g in gs]
    ks = [g[C:2 * C] for g in gs]
    vs = [g[2 * C:3 * C] for g in gs]

    ones_row = jnp.ones((1, HW), f32)
    sqs = [lax.dot_general(q * q, ones_row, (((1,), (1,)), ((), ())),
                           preferred_element_type=f32) for q in qs]
    sks = [lax.dot_general(ones_row, k * k, (((1,), (1,)), ((), ())),
                           preferred_element_type=f32) for k in ks]
    grams = [lax.dot_general(q, k, (((1,), (1,)), ((), ())),
                             preferred_element_type=f32)
             for q, k in zip(qs, ks)]

    eps2 = 1e-24
    hc = C // num_heads
    ri = lax.broadcasted_iota(jnp.int32, (C, C), 0)
    ci = lax.broadcasted_iota(jnp.int32, (C, C), 1)
    same_head = (ri // hc) == (ci // hc)

    ps = []
    for gram, sq, sk in zip(grams, sqs, sks):
        qn = lax.rsqrt(jnp.maximum(sq, eps2))
        kn = lax.rsqrt(jnp.maximum(sk, eps2))
        sc = gram * qn * kn * temp_ref[...]
        sc = jnp.where(same_head, sc, -1e30)
        sc = sc - jnp.max(sc, axis=-1, keepdims=True)
        e = jnp.exp(sc)
        ps.append(e / jnp.sum(e, axis=-1, keepdims=True))

    ms = [lax.dot_general(p, wproj_ref[...], (((0,), (0,)), ((), ())),
                          preferred_element_type=f32) for p in ps]
    for i, (m, v) in enumerate(zip(ms, vs)):
        out = lax.dot_general(m, v, (((0,), (0,)), ((), ())),
                              preferred_element_type=f32)
        o_ref[i] = (out + bproj_ref[...]).reshape(C, H, W)


def _fused_attention_kernel(x_ref, w3_ref, wproj_ref, bproj_ref, temp_ref,
                            o_ref, e0_ref, e1_ref, o0_ref, o1_ref,
                            *, H, W, num_heads):
    bf16 = jnp.bfloat16
    C = x_ref.shape[1]
    HW = H * W
    s = pl.program_id(0)

    pos = lax.broadcasted_iota(jnp.int32, (1, HW), 1)
    jcol = pos % W
    yrow = pos // W

    cl = jnp.where(jcol == 0, 0.0, 1.0).astype(bf16)
    cr = jnp.where(jcol == W - 1, 0.0, 1.0).astype(bf16)
    rt = jnp.where(yrow == 0, 0.0, 1.0).astype(bf16)
    rb = jnp.where(yrow == H - 1, 0.0, 1.0).astype(bf16)

    one = jnp.ones((1, HW), bf16)
    masks = [cl * rt, rt, cr * rt,
             cl, one, cr,
             cl * rb, rb, cr * rb]

    @pl.when(s == 0)
    def _():
        zrow = jnp.zeros((6, HW), bf16)
        indcat = jnp.concatenate(masks + [one, zrow], axis=0)
        for r in (e0_ref, e1_ref, o0_ref, o1_ref):
            r[9 * C:, :] = indcat
        z = jnp.zeros((9 * C, HW), bf16)
        o0_ref[0:9 * C, :] = z
        o1_ref[0:9 * C, :] = z

    bk = dict(C=C, W=W, HW=HW)
    ak = dict(C=C, H=H, W=W, HW=HW, num_heads=num_heads)

    @pl.when(s % 2 == 0)
    def _():
        _attend2(o0_ref, o1_ref, w3_ref, wproj_ref, bproj_ref, temp_ref,
                 o_ref, **ak)
        _build_xcat(x_ref, 0, e0_ref, cl, cr, rt, rb, **bk)
        _build_xcat(x_ref, 1, e1_ref, cl, cr, rt, rb, **bk)

    @pl.when(s % 2 == 1)
    def _():
        _attend2(e0_ref, e1_ref, w3_ref, wproj_ref, bproj_ref, temp_ref,
                 o_ref, **ak)
        _build_xcat(x_ref, 0, o0_ref, cl, cr, rt, rb, **bk)
        _build_xcat(x_ref, 1, o1_ref, cl, cr, rt, rb, **bk)


def kernel(x, temperature, w_qkv, b_qkv, w_dw, b_dw, w_proj, b_proj):
    B, C, H, W = x.shape
    HW = H * W
    num_heads = temperature.shape[0]
    C3 = 3 * C

    wdw9 = w_dw.reshape(9, C3)
    w3 = jnp.einsum('ec,tc->cte', w_qkv, wdw9)
    w3 = w3.reshape(C3, 9 * C)
    wb9 = (b_qkv[None, :] * wdw9).T
    w3_full = jnp.concatenate(
        [w3, wb9, b_dw.reshape(C3, 1), jnp.zeros((C3, 6), w3.dtype)],
        axis=1).astype(jnp.bfloat16)

    bproj_c = b_proj.reshape(C, 1)
    temp_col = jnp.repeat(temperature.astype(jnp.float32),
                          C // num_heads).reshape(C, 1)

    body = functools.partial(_fused_attention_kernel,
                             H=H, W=W, num_heads=num_heads)
    K = 9 * C + 16
    NP = B // 2
    out = pl.pallas_call(
        body,
        out_shape=jax.ShapeDtypeStruct((B, C, H, W), jnp.float32),
        grid=(NP + 1,),
        in_specs=[
            pl.BlockSpec((2, C, H, W),
                         lambda b: (jnp.minimum(b, NP - 1), 0, 0, 0)),
            pl.BlockSpec((C3, K), lambda b: (0, 0)),
            pl.BlockSpec((C, C), lambda b: (0, 0)),
            pl.BlockSpec((C, 1), lambda b: (0, 0)),
            pl.BlockSpec((C, 1), lambda b: (0, 0)),
        ],
        out_specs=pl.BlockSpec((2, C, H, W),
                               lambda b: (jnp.maximum(b - 1, 0), 0, 0, 0)),
        scratch_shapes=[pltpu.VMEM((K, HW), jnp.bfloat16),
                        pltpu.VMEM((K, HW), jnp.bfloat16),
                        pltpu.VMEM((K, HW), jnp.bfloat16),
                        pltpu.VMEM((K, HW), jnp.bfloat16)],
        compiler_params=pltpu.CompilerParams(
            dimension_semantics=("arbitrary",),
            vmem_limit_bytes=64 * 1024 * 1024,
        ),
    )(x, w3_full, w_proj, bproj_c, temp_col)
    return out
